# Initial kernel scaffold; baseline (speedup 1.0000x reference)
import jax
import jax.numpy as jnp
from jax.experimental import pallas as pl


def kernel(flat, cu_seqlens):
    raise NotImplementedError("write your pallas kernel here")


# speedup vs baseline: 1.5760x; 1.5760x over previous
import functools

import jax
import jax.numpy as jnp
from jax import lax
from jax.experimental import pallas as pl
from jax.experimental.pallas import tpu as pltpu
from jax.experimental.pallas import tpu_sc as plsc

_B = 16
_TOTAL = 16384
_D = 1024
_NC = 2
_NS = 16
_NW = _NC * _NS
_RPW = _TOTAL // _NW
_C = 32
_NCHUNK = _RPW // _C
_L = 16


def _sc_partials(flat1d, cnt):
    mesh = plsc.VectorSubcoreMesh(core_axis_name="c", subcore_axis_name="s")

    @functools.partial(
        pl.kernel,
        out_type=jax.ShapeDtypeStruct((_NW, _B * _D), jnp.float32),
        mesh=mesh,
        scratch_types=[
            pltpu.VMEM((_NCHUNK * _B,), jnp.int32),
            pltpu.VMEM((_C * _D,), jnp.float32),
            pltpu.VMEM((_C * _D,), jnp.float32),
            pltpu.VMEM((_B * _D,), jnp.float32),
            pltpu.SemaphoreType.DMA,
            pltpu.SemaphoreType.DMA,
        ],
    )
    def body(flat_hbm, cnt_hbm, out_hbm, cnt_v, buf0, buf1, acc, sem0, sem1):
        wid = lax.axis_index("s") * _NC + lax.axis_index("c")
        row0 = wid * _RPW
        bufs = (buf0, buf1)
        sems = (sem0, sem1)

        def chunk_src(cc):
            off = pl.multiple_of((row0 + cc * _C) * _D, _C * _D)
            return flat_hbm.at[pl.ds(off, _C * _D)]

        pltpu.make_async_copy(chunk_src(0), buf0, sem0).start()
        pltpu.make_async_copy(chunk_src(1), buf1, sem1).start()
        pltpu.sync_copy(cnt_hbm.at[wid], cnt_v)

        zero = jnp.zeros((_L,), jnp.float32)

        def zbody(i, _):
            acc[pl.ds(i * _L, _L)] = zero
            return 0

        lax.fori_loop(0, _B * _D // _L, zbody, 0)

        def process(cc, buf):
            cv = cnt_v[pl.ds(cc * _B, _B)]
            o = jnp.int32(0)
            for j in range(_B):
                n = cv[j]

                @pl.when(n > 0)
                def _():
                    def gbody(gg, _):
                        gbase = j * _D + gg * (4 * _L)
                        a0 = acc[pl.ds(gbase + 0 * _L, _L)]
                        a1 = acc[pl.ds(gbase + 1 * _L, _L)]
                        a2 = acc[pl.ds(gbase + 2 * _L, _L)]
                        a3 = acc[pl.ds(gbase + 3 * _L, _L)]

                        def rbody(r, accs):
                            b = (o + r) * _D + gg * (4 * _L)
                            x0, x1, x2, x3 = accs
                            return (
                                x0 + buf[pl.ds(b + 0 * _L, _L)],
                                x1 + buf[pl.ds(b + 1 * _L, _L)],
                                x2 + buf[pl.ds(b + 2 * _L, _L)],
                                x3 + buf[pl.ds(b + 3 * _L, _L)],
                            )

                        a0, a1, a2, a3 = lax.fori_loop(
                            0, n, rbody, (a0, a1, a2, a3))
                        acc[pl.ds(gbase + 0 * _L, _L)] = a0
                        acc[pl.ds(gbase + 1 * _L, _L)] = a1
                        acc[pl.ds(gbase + 2 * _L, _L)] = a2
                        acc[pl.ds(gbase + 3 * _L, _L)] = a3
                        return 0

                    lax.fori_loop(0, _D // (4 * _L), gbody, 0)

                o = o + n

        def pair(i, _):
            cc = i * 2
            pltpu.make_async_copy(chunk_src(0), buf0, sem0).wait()
            process(cc, buf0)

            @pl.when(cc + 2 < _NCHUNK)
            def _():
                pltpu.make_async_copy(chunk_src(cc + 2), buf0, sem0).start()

            pltpu.make_async_copy(chunk_src(1), buf1, sem1).wait()
            process(cc + 1, buf1)

            @pl.when(cc + 3 < _NCHUNK)
            def _():
                pltpu.make_async_copy(chunk_src(cc + 3), buf1, sem1).start()

            return 0

        lax.fori_loop(0, _NCHUNK // 2, pair, 0)
        pltpu.sync_copy(acc, out_hbm.at[wid])

    return body(flat1d, cnt)


def _tc_reduce(partials):
    def body(p_ref, o_ref):
        o_ref[...] = jnp.sum(p_ref[...], axis=0)

    return pl.pallas_call(
        body,
        out_shape=jax.ShapeDtypeStruct((_B, _D), jnp.float32),
    )(partials)


def kernel(flat, cu_seqlens):
    cu = cu_seqlens.astype(jnp.int32)
    base = (jnp.arange(_NW * _NCHUNK, dtype=jnp.int32) * _C)[:, None]
    lo = jnp.maximum(cu[:-1][None, :], base)
    hi = jnp.minimum(cu[1:][None, :], base + _C)
    cnt = jnp.maximum(hi - lo, 0).astype(jnp.int32).reshape(_NW, _NCHUNK * _B)

    partials = _sc_partials(flat.reshape(-1), cnt)
    return _tc_reduce(partials.reshape(_NW, _B, _D))


# speedup vs baseline: 1.7120x; 1.0863x over previous
import functools

import jax
import jax.numpy as jnp
from jax import lax
from jax.experimental import pallas as pl
from jax.experimental.pallas import tpu as pltpu
from jax.experimental.pallas import tpu_sc as plsc

_B = 16
_TOTAL = 16384
_D = 1024
_NC = 2
_NS = 16
_NW = _NC * _NS
_RPW = _TOTAL // _NW
_C = 32
_NCHUNK = _RPW // _C
_L = 16


def _sc_partials(flat1d, cnt):
    mesh = plsc.VectorSubcoreMesh(core_axis_name="c", subcore_axis_name="s")

    @functools.partial(
        pl.kernel,
        out_type=jax.ShapeDtypeStruct((_NW, _B * _D), jnp.float32),
        mesh=mesh,
        scratch_types=[
            pltpu.VMEM((_NCHUNK * _B,), jnp.int32),
            pltpu.VMEM((_C * _D,), jnp.float32),
            pltpu.VMEM((_C * _D,), jnp.float32),
            pltpu.VMEM((_B * _D,), jnp.float32),
            pltpu.SemaphoreType.DMA,
            pltpu.SemaphoreType.DMA,
        ],
    )
    def body(flat_hbm, cnt_hbm, out_hbm, cnt_v, buf0, buf1, acc, sem0, sem1):
        wid = lax.axis_index("s") * _NC + lax.axis_index("c")
        row0 = wid * _RPW
        bufs = (buf0, buf1)
        sems = (sem0, sem1)

        def chunk_src(cc):
            off = pl.multiple_of((row0 + cc * _C) * _D, _C * _D)
            return flat_hbm.at[pl.ds(off, _C * _D)]

        pltpu.make_async_copy(chunk_src(0), buf0, sem0).start()
        pltpu.make_async_copy(chunk_src(1), buf1, sem1).start()
        pltpu.sync_copy(cnt_hbm.at[wid], cnt_v)

        zero = jnp.zeros((_L,), jnp.float32)

        def zbody(i, _):
            acc[pl.ds(i * _L, _L)] = zero
            return 0

        lax.fori_loop(0, _B * _D // _L, zbody, 0)

        def process(cc, buf):
            cv = cnt_v[pl.ds(cc * _B, _B)]
            ns = [cv[j] for j in range(_B)]
            is_full = ns[0] == _C
            s = jnp.int32(0)
            for j in range(_B):
                fj = ns[j] == _C
                is_full = jnp.logical_or(is_full, fj)
                s = s + jnp.where(fj, jnp.int32(j), jnp.int32(0))

            @pl.when(is_full)
            def _():
                def gbody(gg, _):
                    gbase = s * _D + gg * (4 * _L)
                    a0 = acc[pl.ds(gbase + 0 * _L, _L)]
                    a1 = acc[pl.ds(gbase + 1 * _L, _L)]
                    a2 = acc[pl.ds(gbase + 2 * _L, _L)]
                    a3 = acc[pl.ds(gbase + 3 * _L, _L)]
                    for r in range(_C):
                        b = r * _D + gg * (4 * _L)
                        a0 = a0 + buf[pl.ds(b + 0 * _L, _L)]
                        a1 = a1 + buf[pl.ds(b + 1 * _L, _L)]
                        a2 = a2 + buf[pl.ds(b + 2 * _L, _L)]
                        a3 = a3 + buf[pl.ds(b + 3 * _L, _L)]
                    acc[pl.ds(gbase + 0 * _L, _L)] = a0
                    acc[pl.ds(gbase + 1 * _L, _L)] = a1
                    acc[pl.ds(gbase + 2 * _L, _L)] = a2
                    acc[pl.ds(gbase + 3 * _L, _L)] = a3
                    return 0

                lax.fori_loop(0, _D // (4 * _L), gbody, 0)

            @pl.when(jnp.logical_not(is_full))
            def _():
                _process_runs(ns, buf)

        def _process_runs(ns, buf):
            o = jnp.int32(0)
            for j in range(_B):
                n = ns[j]

                @pl.when(n > 0)
                def _():
                    def gbody(gg, _):
                        gbase = j * _D + gg * (4 * _L)
                        a0 = acc[pl.ds(gbase + 0 * _L, _L)]
                        a1 = acc[pl.ds(gbase + 1 * _L, _L)]
                        a2 = acc[pl.ds(gbase + 2 * _L, _L)]
                        a3 = acc[pl.ds(gbase + 3 * _L, _L)]

                        def rbody(r, accs):
                            b = (o + r) * _D + gg * (4 * _L)
                            x0, x1, x2, x3 = accs
                            return (
                                x0 + buf[pl.ds(b + 0 * _L, _L)],
                                x1 + buf[pl.ds(b + 1 * _L, _L)],
                                x2 + buf[pl.ds(b + 2 * _L, _L)],
                                x3 + buf[pl.ds(b + 3 * _L, _L)],
                            )

                        a0, a1, a2, a3 = lax.fori_loop(
                            0, n, rbody, (a0, a1, a2, a3))
                        acc[pl.ds(gbase + 0 * _L, _L)] = a0
                        acc[pl.ds(gbase + 1 * _L, _L)] = a1
                        acc[pl.ds(gbase + 2 * _L, _L)] = a2
                        acc[pl.ds(gbase + 3 * _L, _L)] = a3
                        return 0

                    lax.fori_loop(0, _D // (4 * _L), gbody, 0)

                o = o + n

        def pair(i, _):
            cc = i * 2
            pltpu.make_async_copy(chunk_src(0), buf0, sem0).wait()
            process(cc, buf0)

            @pl.when(cc + 2 < _NCHUNK)
            def _():
                pltpu.make_async_copy(chunk_src(cc + 2), buf0, sem0).start()

            pltpu.make_async_copy(chunk_src(1), buf1, sem1).wait()
            process(cc + 1, buf1)

            @pl.when(cc + 3 < _NCHUNK)
            def _():
                pltpu.make_async_copy(chunk_src(cc + 3), buf1, sem1).start()

            return 0

        lax.fori_loop(0, _NCHUNK // 2, pair, 0)
        pltpu.sync_copy(acc, out_hbm.at[wid])

    return body(flat1d, cnt)


def _tc_reduce(partials):
    def body(p_ref, o_ref):
        o_ref[...] = jnp.sum(p_ref[...], axis=0)

    return pl.pallas_call(
        body,
        out_shape=jax.ShapeDtypeStruct((_B, _D), jnp.float32),
    )(partials)


def kernel(flat, cu_seqlens):
    cu = cu_seqlens.astype(jnp.int32)
    base = (jnp.arange(_NW * _NCHUNK, dtype=jnp.int32) * _C)[:, None]
    lo = jnp.maximum(cu[:-1][None, :], base)
    hi = jnp.minimum(cu[1:][None, :], base + _C)
    cnt = jnp.maximum(hi - lo, 0).astype(jnp.int32).reshape(_NW, _NCHUNK * _B)

    partials = _sc_partials(flat.reshape(-1), cnt)
    return _tc_reduce(partials.reshape(_NW, _B, _D))


# speedup vs baseline: 3.2292x; 1.8863x over previous
import functools

import jax
import jax.numpy as jnp
from jax import lax
from jax.experimental import pallas as pl
from jax.experimental.pallas import tpu as pltpu
from jax.experimental.pallas import tpu_sc as plsc

_B = 16
_TOTAL = 16384
_D = 1024
_NC = 2
_NS = 16
_NW = _NC * _NS
_RPW = _TOTAL // _NW
_C = 32
_NCHUNK = _RPW // _C
_L = 16


def _sc_partials(flat, cnt):
    mesh = plsc.VectorSubcoreMesh(core_axis_name="c", subcore_axis_name="s")

    @functools.partial(
        pl.kernel,
        out_type=jax.ShapeDtypeStruct((_NW, _B, _D), jnp.float32),
        mesh=mesh,
        scratch_types=[
            pltpu.VMEM((_NCHUNK * _B,), jnp.int32),
            pltpu.VMEM((_C, _D), jnp.float32),
            pltpu.VMEM((_C, _D), jnp.float32),
            pltpu.VMEM((_B, _D), jnp.float32),
            pltpu.SemaphoreType.DMA,
            pltpu.SemaphoreType.DMA,
        ],
    )
    def body(flat_hbm, cnt_hbm, out_hbm, cnt_v, buf0, buf1, acc, sem0, sem1):
        wid = lax.axis_index("s") * _NC + lax.axis_index("c")
        row0 = wid * _RPW

        def chunk_src(cc):
            return flat_hbm.at[pl.ds(pl.multiple_of(row0 + cc * _C, _C), _C), :]

        pltpu.make_async_copy(chunk_src(0), buf0, sem0).start()
        pltpu.make_async_copy(chunk_src(1), buf1, sem1).start()
        pltpu.sync_copy(cnt_hbm.at[wid], cnt_v)

        zero = jnp.zeros((_L,), jnp.float32)

        for j in range(_B):
            def zbody(g, _, j=j):
                acc[j, pl.ds(g * _L, _L)] = zero
                return 0

            lax.fori_loop(0, _D // _L, zbody, 0)

        def process(cc, buf):
            cv = cnt_v[pl.ds(cc * _B, _B)]
            ns = [cv[j] for j in range(_B)]
            is_full = ns[0] == _C
            s = jnp.int32(0)
            for j in range(_B):
                fj = ns[j] == _C
                is_full = jnp.logical_or(is_full, fj)
                s = s + jnp.where(fj, jnp.int32(j), jnp.int32(0))

            @pl.when(is_full)
            def _():
                def gbody(gg, _):
                    gbase = gg * (4 * _L)
                    a0 = acc[s, pl.ds(gbase + 0 * _L, _L)]
                    a1 = acc[s, pl.ds(gbase + 1 * _L, _L)]
                    a2 = acc[s, pl.ds(gbase + 2 * _L, _L)]
                    a3 = acc[s, pl.ds(gbase + 3 * _L, _L)]
                    for r in range(_C):
                        a0 = a0 + buf[r, pl.ds(gbase + 0 * _L, _L)]
                        a1 = a1 + buf[r, pl.ds(gbase + 1 * _L, _L)]
                        a2 = a2 + buf[r, pl.ds(gbase + 2 * _L, _L)]
                        a3 = a3 + buf[r, pl.ds(gbase + 3 * _L, _L)]
                    acc[s, pl.ds(gbase + 0 * _L, _L)] = a0
                    acc[s, pl.ds(gbase + 1 * _L, _L)] = a1
                    acc[s, pl.ds(gbase + 2 * _L, _L)] = a2
                    acc[s, pl.ds(gbase + 3 * _L, _L)] = a3
                    return 0

                lax.fori_loop(0, _D // (4 * _L), gbody, 0)

            @pl.when(jnp.logical_not(is_full))
            def _():
                _process_runs(ns, buf)

        def _process_runs(ns, buf):
            o = jnp.int32(0)
            for j in range(_B):
                n = ns[j]

                @pl.when(n > 0)
                def _(j=j, n=n, o=o):
                    def gbody(gg, _):
                        gbase = gg * (4 * _L)
                        a0 = acc[j, pl.ds(gbase + 0 * _L, _L)]
                        a1 = acc[j, pl.ds(gbase + 1 * _L, _L)]
                        a2 = acc[j, pl.ds(gbase + 2 * _L, _L)]
                        a3 = acc[j, pl.ds(gbase + 3 * _L, _L)]

                        def rbody(r, accs):
                            x0, x1, x2, x3 = accs
                            return (
                                x0 + buf[o + r, pl.ds(gbase + 0 * _L, _L)],
                                x1 + buf[o + r, pl.ds(gbase + 1 * _L, _L)],
                                x2 + buf[o + r, pl.ds(gbase + 2 * _L, _L)],
                                x3 + buf[o + r, pl.ds(gbase + 3 * _L, _L)],
                            )

                        a0, a1, a2, a3 = lax.fori_loop(
                            0, n, rbody, (a0, a1, a2, a3))
                        acc[j, pl.ds(gbase + 0 * _L, _L)] = a0
                        acc[j, pl.ds(gbase + 1 * _L, _L)] = a1
                        acc[j, pl.ds(gbase + 2 * _L, _L)] = a2
                        acc[j, pl.ds(gbase + 3 * _L, _L)] = a3
                        return 0

                    lax.fori_loop(0, _D // (4 * _L), gbody, 0)

                o = o + n

        def pair(i, _):
            cc = i * 2
            pltpu.make_async_copy(chunk_src(0), buf0, sem0).wait()
            process(cc, buf0)

            @pl.when(cc + 2 < _NCHUNK)
            def _():
                pltpu.make_async_copy(chunk_src(cc + 2), buf0, sem0).start()

            pltpu.make_async_copy(chunk_src(1), buf1, sem1).wait()
            process(cc + 1, buf1)

            @pl.when(cc + 3 < _NCHUNK)
            def _():
                pltpu.make_async_copy(chunk_src(cc + 3), buf1, sem1).start()

            return 0

        lax.fori_loop(0, _NCHUNK // 2, pair, 0)
        pltpu.sync_copy(acc, out_hbm.at[wid])

    return body(flat, cnt)


def _tc_reduce(partials):
    def body(p_ref, o_ref):
        o_ref[...] = jnp.sum(p_ref[...], axis=0)

    return pl.pallas_call(
        body,
        out_shape=jax.ShapeDtypeStruct((_B, _D), jnp.float32),
    )(partials)


def kernel(flat, cu_seqlens):
    cu = cu_seqlens.astype(jnp.int32)
    base = (jnp.arange(_NW * _NCHUNK, dtype=jnp.int32) * _C)[:, None]
    lo = jnp.maximum(cu[:-1][None, :], base)
    hi = jnp.minimum(cu[1:][None, :], base + _C)
    cnt = jnp.maximum(hi - lo, 0).astype(jnp.int32).reshape(_NW, _NCHUNK * _B)

    partials = _sc_partials(flat, cnt)
    return _tc_reduce(partials)
